# baseline (speedup 1.0000x reference)
import jax
import jax.numpy as jnp
import numpy as np
from jax import lax
from jax.experimental import pallas as pl
from jax.experimental.pallas import tpu as pltpu
from jax.experimental.pallas import tpu_sc as plsc

_MAXA = 100
_T = 1000
_WIN = 3888


def _np_rotl(x, r):
    return (x << np.uint32(r)) | (x >> np.uint32(32 - r))


def _np_threefry(k0, k1, x0, x1):
    ks = [np.uint32(k0), np.uint32(k1),
          np.uint32(k0) ^ np.uint32(k1) ^ np.uint32(0x1BD11BDA)]
    x0 = x0 + ks[0]
    x1 = x1 + ks[1]
    rots = [[13, 15, 26, 6], [17, 29, 16, 24]]
    for i in range(5):
        for r in rots[i % 2]:
            x0 = x0 + x1
            x1 = _np_rotl(x1, r)
            x1 = x0 ^ x1
        x0 = x0 + ks[(i + 1) % 3]
        x1 = x1 + ks[(i + 2) % 3] + np.uint32(i + 1)
    return x0, x1


def _np_split(kd, n):
    idx = np.arange(n, dtype=np.uint32)
    with np.errstate(over="ignore"):
        a, b = _np_threefry(kd[0], kd[1], np.zeros(n, np.uint32), idx)
    return np.stack([a, b], axis=1)


def _np_bits(kd, n):
    idx = np.arange(n, dtype=np.uint32)
    with np.errstate(over="ignore"):
        a, b = _np_threefry(kd[0], kd[1], np.zeros(n, np.uint32), idx)
    return a ^ b


def _np_randint(kd, n, minval, maxval):
    k1, k2 = _np_split(kd, 2)
    hi, lo = _np_bits(k1, n), _np_bits(k2, n)
    span = np.uint32(maxval - minval)
    mult = np.uint32((int(2 ** 16) % int(span)) ** 2 % int(span))
    with np.errstate(over="ignore"):
        off = ((hi % span) * mult + lo % span) % span
    return (minval + off.astype(np.int32)).astype(np.int32)


_KEYS = _np_split(np.array([0, 1], np.uint32), 3)
_TVALS = _np_randint(_KEYS[0], 16, 1, _T)
_SVALS = _np_randint(_KEYS[1], 16, 1, _T)
_KE0 = int(_KEYS[2][0].view(np.int32))
_KE1 = int(_KEYS[2][1].view(np.int32))
_SELR = np.zeros((16, 8), np.float32)
_SELR[np.arange(16), _SVALS // 128] = 1.0
_SELC = np.zeros((16, 128), np.float32)
_SELC[np.arange(16), _SVALS % 128] = 1.0

_LO = np.nextafter(np.float32(-1), np.float32(0)).astype(np.float32)
_SPAN = (np.float32(1.0) - _LO).astype(np.float32)
_SQRT2 = np.float32(np.sqrt(2.0))


def _gen_eps_kernel(out_ref):
    rows, cols = out_ref.shape
    k0 = _KE0
    k1 = _KE1
    ks = (k0, k1, k0 ^ k1 ^ 0x1BD11BDA)
    rio = lax.broadcasted_iota(jnp.int32, (rows, cols), 0)
    cio = lax.broadcasted_iota(jnp.int32, (rows, cols), 1)
    idx = rio * cols + cio
    x0 = jnp.zeros((rows, cols), jnp.int32) + k0
    x1 = idx + k1
    rots = ((13, 15, 26, 6), (17, 29, 16, 24))
    for g in range(5):
        for r_ in rots[g % 2]:
            x0 = x0 + x1
            x1 = lax.shift_left(x1, r_) | lax.shift_right_logical(x1, 32 - r_)
            x1 = x0 ^ x1
        x0 = x0 + ks[(g + 1) % 3]
        x1 = x1 + ks[(g + 2) % 3] + (g + 1)
    bits = x0 ^ x1
    fb = lax.shift_right_logical(bits, 9) | 0x3F800000
    u = lax.bitcast_convert_type(fb, jnp.float32) - 1.0
    u = jnp.maximum(_LO, u * _SPAN + _LO)
    w = -jnp.log((1.0 - u) * (1.0 + u))
    wc = w - 2.5
    pc = jnp.full_like(w, 2.81022636e-08)
    for c in (3.43273939e-07, -3.5233877e-06, -4.39150654e-06, 0.00021858087,
              -0.00125372503, -0.00417768164, 0.246640727, 1.50140941):
        pc = jnp.float32(c) + pc * wc
    wt = jnp.sqrt(w) - 3.0
    qt = jnp.full_like(w, -0.000200214257)
    for c in (0.000100950558, 0.00134934322, -0.00367342844, 0.00573950773,
              -0.0076224613, 0.00943887047, 1.00167406, 2.83297682):
        qt = jnp.float32(c) + qt * wt
    poly = jnp.where(w < 5.0, pc, qt)
    out_ref[...] = (_SQRT2 * poly) * u


def _sc_stats_body(starts3_hbm, ends3_hbm, t_hbm, s_hbm, al_hbm, si_hbm,
                   eps_hbm, out_hbm, scal_hbm,
                   st_v, en_v, t_v, s_v, al_v, si_v, win_v, row_v, scal_v):
    h = lax.axis_index("c")
    sid = lax.axis_index("s")
    wid = h * 16 + sid
    pltpu.sync_copy(starts3_hbm, st_v)
    pltpu.sync_copy(ends3_hbm, en_v)
    lo_seg = st_v[pl.ds(sid, 16)][0]
    hi_seg = en_v[pl.ds(sid, 16)][0]
    mid = (lo_seg + hi_seg) // 2
    lo = jnp.where(h == 0, lo_seg, mid)
    hi = jnp.where(h == 0, mid, hi_seg)
    lo8 = jnp.minimum((lo // 8) * 8, eps_hbm.shape[0] - _WIN)
    pltpu.sync_copy(eps_hbm.at[pl.ds(lo8, _WIN)], win_v)
    io = lax.iota(jnp.int32, 16)
    zero = jnp.zeros((16,), jnp.float32)

    def body(k, accs):
        base = k * 48
        out = []
        for q in range(3):
            off = base + q * 16
            v = win_v[pl.ds(off, 16)]
            pos = (lo8 + off) + io
            msk = (pos >= lo) & (pos < hi)
            out.append(accs[q] + jnp.where(msk, v, 0.0))
        return tuple(out)

    accs = lax.fori_loop(0, _WIN // 48, body, (zero, zero, zero))
    for q in range(3):
        row_v[q, :] = accs[q]
    for q in range(3, 8):
        row_v[q, :] = zero
    pltpu.sync_copy(row_v, out_hbm.at[pl.ds(8 * wid, 8)])

    @pl.when(wid == 0)
    def _scal():
        pltpu.sync_copy(t_hbm, t_v)
        pltpu.sync_copy(s_hbm, s_v)
        pltpu.sync_copy(al_hbm, al_v)
        pltpu.sync_copy(si_hbm, si_v)
        scal_v[0, :] = plsc.load_gather(al_v, [t_v[...]])
        scal_v[1, :] = plsc.load_gather(si_v, [s_v[...]])
        pltpu.sync_copy(scal_v, scal_hbm)


def _as_kernel(starts_ref, ends_ref, selr_ref, selc_ref, si8_ref, z_ref,
               comp_ref, as_ref, sig_ref):
    i = pl.program_id(0)
    blk = z_ref.shape[0]

    @pl.when(i == 0)
    def _build():
        x = jnp.dot(selr_ref[...], si8_ref[...],
                    precision=lax.Precision.HIGHEST,
                    preferred_element_type=jnp.float32)
        sig_ref[...] = jnp.sum(x * selc_ref[...], axis=1, keepdims=True)

    rows = lax.broadcasted_iota(jnp.int32, (blk, 16), 0) + i * blk
    oh = ((rows >= starts_ref[...]) & (rows < ends_ref[...])).astype(jnp.float32)
    sigrow = jnp.dot(oh, sig_ref[...], precision=lax.Precision.HIGHEST,
                     preferred_element_type=jnp.float32)
    cio = lax.broadcasted_iota(jnp.int32, (blk, _MAXA), 1)
    as_ref[...] = ((cio == (z_ref[...] - 1)).astype(jnp.float32)
                   + comp_ref[...] * sigrow)


def _main_kernel(starts_ref, ends_ref, inv_na_ref, phq_ref,
                 sums_ref, scal_ref, r_ref, eps_ref,
                 rt_ref, epsr_ref, table_ref):
    i = pl.program_id(0)
    blk = r_ref.shape[0]

    @pl.when(i == 0)
    def _build():
        sums = sums_ref[...]
        lane = lax.broadcasted_iota(jnp.int32, (256, 16), 1)
        ph = jnp.remainder(phq_ref[...] + lane, 3)
        rio = lax.broadcasted_iota(jnp.int32, (16, 256), 1)
        seg = jnp.remainder(rio // 8, 16)
        sio = lax.broadcasted_iota(jnp.int32, (16, 256), 0)
        smat = (seg == sio).astype(jnp.float32)
        cols = []
        for c in range(3):
            xc = jnp.where(ph == c, sums, 0.0)
            scc = jnp.sum(xc, axis=1, keepdims=True)
            cols.append(jnp.dot(smat, scc, precision=lax.Precision.HIGHEST,
                                preferred_element_type=jnp.float32))
        mean = jnp.concatenate(cols, axis=1) * inv_na_ref[...]
        eye = (lax.broadcasted_iota(jnp.int32, (16, 16), 0)
               == lax.broadcasted_iota(jnp.int32, (16, 16), 1)).astype(jnp.float32)
        asig = lax.dot_general(eye, scal_ref[...], (((1,), (1,)), ((), ())),
                               precision=lax.Precision.HIGHEST,
                               preferred_element_type=jnp.float32)
        alpha = asig[:, 0:1]
        table_ref[...] = jnp.concatenate(
            [jnp.sqrt(alpha), jnp.sqrt(1.0 - alpha), asig[:, 1:2], mean,
             jnp.zeros((16, 2), jnp.float32)], axis=1)

    rows = lax.broadcasted_iota(jnp.int32, (blk, 16), 0) + i * blk
    oh = ((rows >= starts_ref[...]) & (rows < ends_ref[...])).astype(jnp.float32)
    vals = jnp.dot(oh, table_ref[...], precision=lax.Precision.HIGHEST,
                   preferred_element_type=jnp.float32)
    epsr = eps_ref[...] - vals[:, 3:6]
    rt_ref[...] = vals[:, 0:1] * r_ref[...] + vals[:, 1:2] * epsr
    epsr_ref[...] = epsr


def kernel(r, Z, composition_probs, num_atoms, alphas, type_sigmas):
    N = r.shape[0]
    B = num_atoms.shape[0]
    t = jnp.asarray(_TVALS)
    s = jnp.asarray(_SVALS)
    BLK = 2048
    nb = N // BLK
    eps_flat = pl.pallas_call(
        _gen_eps_kernel,
        out_shape=jax.ShapeDtypeStruct((3 * N // 128, 128), jnp.float32),
    )()
    eps = eps_flat.reshape(N, 3)

    ends = jnp.cumsum(num_atoms, dtype=jnp.int32)
    starts = ends - num_atoms
    starts_row = starts.reshape(1, B)
    ends_row = ends.reshape(1, B)
    inv_na_col = (1.0 / num_atoms.astype(jnp.float32)).reshape(B, 1)
    al_pad = jnp.pad(alphas, (0, 1024 - _T))
    si_pad = jnp.pad(type_sigmas, (0, 1024 - _T))
    starts3 = 3 * starts
    ends3 = 3 * ends
    starts3_p = jnp.pad(starts3, (0, 16))
    ends3_p = jnp.pad(ends3, (0, 16))

    mid3 = (starts3 + ends3) // 2
    lo_w = jnp.concatenate([starts3, mid3])
    lo8_w = jnp.minimum((lo_w // 8) * 8, 3 * N - _WIN)
    qoff = jnp.arange(8, dtype=jnp.int32).reshape(1, 8)
    phq = jnp.remainder(lo8_w.reshape(32, 1) + 16 * qoff, 3).reshape(256, 1)

    mesh = plsc.VectorSubcoreMesh(core_axis_name="c", subcore_axis_name="s")
    sc_stats = pl.kernel(
        _sc_stats_body,
        out_type=(jax.ShapeDtypeStruct((256, 16), jnp.float32),
                  jax.ShapeDtypeStruct((2, 16), jnp.float32)),
        mesh=mesh,
        compiler_params=pltpu.CompilerParams(needs_layout_passes=False),
        scratch_types=[
            pltpu.VMEM((32,), jnp.int32),
            pltpu.VMEM((32,), jnp.int32),
            pltpu.VMEM((16,), jnp.int32),
            pltpu.VMEM((16,), jnp.int32),
            pltpu.VMEM((1024,), jnp.float32),
            pltpu.VMEM((1024,), jnp.float32),
            pltpu.VMEM((_WIN,), jnp.float32),
            pltpu.VMEM((8, 16), jnp.float32),
            pltpu.VMEM((2, 16), jnp.float32),
        ],
    )
    sums, scal = sc_stats(starts3_p, ends3_p, t, s, al_pad, si_pad,
                          eps_flat.reshape(-1))

    si8 = jnp.pad(type_sigmas, (0, 1024 - _T)).reshape(8, 128)
    a_s = pl.pallas_call(
        _as_kernel,
        grid=(nb,),
        in_specs=[
            pl.BlockSpec((1, B), lambda i: (0, 0)),
            pl.BlockSpec((1, B), lambda i: (0, 0)),
            pl.BlockSpec((B, 8), lambda i: (0, 0)),
            pl.BlockSpec((B, 128), lambda i: (0, 0)),
            pl.BlockSpec((8, 128), lambda i: (0, 0)),
            pl.BlockSpec((BLK, 1), lambda i: (i, 0)),
            pl.BlockSpec((BLK, _MAXA), lambda i: (i, 0)),
        ],
        out_specs=pl.BlockSpec((BLK, _MAXA), lambda i: (i, 0)),
        out_shape=jax.ShapeDtypeStruct((N, _MAXA), jnp.float32),
        scratch_shapes=[pltpu.VMEM((B, 1), jnp.float32)],
    )(starts_row, ends_row, jnp.asarray(_SELR), jnp.asarray(_SELC), si8,
      Z.reshape(N, 1), composition_probs)

    rt, epsr = pl.pallas_call(
        _main_kernel,
        grid=(nb,),
        in_specs=[
            pl.BlockSpec((1, B), lambda i: (0, 0)),
            pl.BlockSpec((1, B), lambda i: (0, 0)),
            pl.BlockSpec((B, 1), lambda i: (0, 0)),
            pl.BlockSpec((256, 1), lambda i: (0, 0)),
            pl.BlockSpec((256, 16), lambda i: (0, 0)),
            pl.BlockSpec((2, 16), lambda i: (0, 0)),
            pl.BlockSpec((BLK, 3), lambda i: (i, 0)),
            pl.BlockSpec((BLK, 3), lambda i: (i, 0)),
        ],
        out_specs=[
            pl.BlockSpec((BLK, 3), lambda i: (i, 0)),
            pl.BlockSpec((BLK, 3), lambda i: (i, 0)),
        ],
        out_shape=[
            jax.ShapeDtypeStruct((N, 3), jnp.float32),
            jax.ShapeDtypeStruct((N, 3), jnp.float32),
        ],
        scratch_shapes=[pltpu.VMEM((B, 8), jnp.float32)],
    )(starts_row, ends_row, inv_na_col, phq, sums, scal, r, eps)

    return rt, a_s, epsr, t[:, None], s[:, None]


# speedup vs baseline: 3.2459x; 3.2459x over previous
import jax
import jax.numpy as jnp
import numpy as np
from jax import lax
from jax.experimental import pallas as pl
from jax.experimental.pallas import tpu as pltpu
from jax.experimental.pallas import tpu_sc as plsc

_MAXA = 100
_T = 1000
_WIN = 1296


def _np_rotl(x, r):
    return (x << np.uint32(r)) | (x >> np.uint32(32 - r))


def _np_threefry(k0, k1, x0, x1):
    ks = [np.uint32(k0), np.uint32(k1),
          np.uint32(k0) ^ np.uint32(k1) ^ np.uint32(0x1BD11BDA)]
    x0 = x0 + ks[0]
    x1 = x1 + ks[1]
    rots = [[13, 15, 26, 6], [17, 29, 16, 24]]
    for i in range(5):
        for r in rots[i % 2]:
            x0 = x0 + x1
            x1 = _np_rotl(x1, r)
            x1 = x0 ^ x1
        x0 = x0 + ks[(i + 1) % 3]
        x1 = x1 + ks[(i + 2) % 3] + np.uint32(i + 1)
    return x0, x1


def _np_split(kd, n):
    idx = np.arange(n, dtype=np.uint32)
    with np.errstate(over="ignore"):
        a, b = _np_threefry(kd[0], kd[1], np.zeros(n, np.uint32), idx)
    return np.stack([a, b], axis=1)


def _np_bits(kd, n):
    idx = np.arange(n, dtype=np.uint32)
    with np.errstate(over="ignore"):
        a, b = _np_threefry(kd[0], kd[1], np.zeros(n, np.uint32), idx)
    return a ^ b


def _np_randint(kd, n, minval, maxval):
    k1, k2 = _np_split(kd, 2)
    hi, lo = _np_bits(k1, n), _np_bits(k2, n)
    span = np.uint32(maxval - minval)
    mult = np.uint32((int(2 ** 16) % int(span)) ** 2 % int(span))
    with np.errstate(over="ignore"):
        off = ((hi % span) * mult + lo % span) % span
    return (minval + off.astype(np.int32)).astype(np.int32)


_KEYS = _np_split(np.array([0, 1], np.uint32), 3)
_TVALS = _np_randint(_KEYS[0], 16, 1, _T)
_SVALS = _np_randint(_KEYS[1], 16, 1, _T)
_KE0 = int(_KEYS[2][0].view(np.int32))
_KE1 = int(_KEYS[2][1].view(np.int32))

_LO = np.nextafter(np.float32(-1), np.float32(0)).astype(np.float32)
_SPAN = (np.float32(1.0) - _LO).astype(np.float32)
_SQRT2 = np.float32(np.sqrt(2.0))


def _gen_eps_kernel(out_ref):
    rows, cols = out_ref.shape
    n_atoms = rows * cols // 3
    k0 = _KE0
    k1 = _KE1
    ks = (k0, k1, k0 ^ k1 ^ 0x1BD11BDA)
    rio = lax.broadcasted_iota(jnp.int32, (rows, cols), 0)
    cio = lax.broadcasted_iota(jnp.int32, (rows, cols), 1)
    p = rio * cols + cio
    idx = (p % n_atoms) * 3 + p // n_atoms
    x0 = jnp.zeros((rows, cols), jnp.int32) + k0
    x1 = idx + k1
    rots = ((13, 15, 26, 6), (17, 29, 16, 24))
    for g in range(5):
        for r_ in rots[g % 2]:
            x0 = x0 + x1
            x1 = lax.shift_left(x1, r_) | lax.shift_right_logical(x1, 32 - r_)
            x1 = x0 ^ x1
        x0 = x0 + ks[(g + 1) % 3]
        x1 = x1 + ks[(g + 2) % 3] + (g + 1)
    bits = x0 ^ x1
    fb = lax.shift_right_logical(bits, 9) | 0x3F800000
    u = lax.bitcast_convert_type(fb, jnp.float32) - 1.0
    u = jnp.maximum(_LO, u * _SPAN + _LO)
    w = -jnp.log((1.0 - u) * (1.0 + u))
    wc = w - 2.5
    pc = jnp.full_like(w, 2.81022636e-08)
    for c in (3.43273939e-07, -3.5233877e-06, -4.39150654e-06, 0.00021858087,
              -0.00125372503, -0.00417768164, 0.246640727, 1.50140941):
        pc = jnp.float32(c) + pc * wc
    wt = jnp.sqrt(w) - 3.0
    qt = jnp.full_like(w, -0.000200214257)
    for c in (0.000100950558, 0.00134934322, -0.00367342844, 0.00573950773,
              -0.0076224613, 0.00943887047, 1.00167406, 2.83297682):
        qt = jnp.float32(c) + qt * wt
    poly = jnp.where(w < 5.0, pc, qt)
    out_ref[...] = (_SQRT2 * poly) * u


def _sc_stats_body(starts_hbm, ends_hbm, t_hbm, s_hbm, al_hbm, si_hbm,
                   eps_hbm, out_hbm, scal_hbm,
                   st_v, en_v, t_v, s_v, al_v, si_v, win_v, row_v, scal_v):
    h = lax.axis_index("c")
    sid = lax.axis_index("s")
    wid = h * 16 + sid
    n_atoms = eps_hbm.shape[0] // 3
    pltpu.sync_copy(starts_hbm, st_v)
    pltpu.sync_copy(ends_hbm, en_v)
    lo_seg = st_v[pl.ds(sid, 16)][0]
    hi_seg = en_v[pl.ds(sid, 16)][0]
    mid = (lo_seg + hi_seg) // 2
    lo = jnp.where(h == 0, lo_seg, mid)
    hi = jnp.where(h == 0, mid, hi_seg)
    lo8 = jnp.minimum((lo // 8) * 8, n_atoms - _WIN)
    io = lax.iota(jnp.int32, 16)
    zero = jnp.zeros((16,), jnp.float32)
    for c in range(3):
        pltpu.sync_copy(eps_hbm.at[pl.ds(c * n_atoms + lo8, _WIN)], win_v)

        def body(k, acc):
            off = k * 16
            v = win_v[pl.ds(off, 16)]
            pos = (lo8 + off) + io
            msk = (pos >= lo) & (pos < hi)
            return acc + jnp.where(msk, v, 0.0)

        acc = lax.fori_loop(0, _WIN // 16, body, zero)
        row_v[c, :] = acc
    for c in range(3, 8):
        row_v[c, :] = zero
    pltpu.sync_copy(row_v, out_hbm.at[pl.ds(8 * wid, 8)])

    @pl.when(wid == 0)
    def _scal():
        pltpu.sync_copy(t_hbm, t_v)
        pltpu.sync_copy(s_hbm, s_v)
        pltpu.sync_copy(al_hbm, al_v)
        pltpu.sync_copy(si_hbm, si_v)
        scal_v[0, :] = plsc.load_gather(al_v, [t_v[...]])
        scal_v[1, :] = plsc.load_gather(si_v, [s_v[...]])
        pltpu.sync_copy(scal_v, scal_hbm)


def _main_kernel(starts_ref, ends_ref, inv_na_ref,
                 sums_ref, scal_ref, r_ref, eps_ref, z_ref, comp_ref,
                 rt_ref, epsr_ref, as_ref, table_ref):
    i = pl.program_id(0)
    blk = r_ref.shape[1]

    @pl.when(i == 0)
    def _build():
        rowsum = jnp.dot(sums_ref[...], jnp.ones((16, 1), jnp.float32),
                         precision=lax.Precision.HIGHEST,
                         preferred_element_type=jnp.float32)
        rio = lax.broadcasted_iota(jnp.int32, (16, 256), 1)
        sio = lax.broadcasted_iota(jnp.int32, (16, 256), 0)
        wseg = jnp.remainder(rio // 8, 16)
        cols = []
        for c in range(3):
            smat = ((wseg == sio) & (rio % 8 == c)).astype(jnp.float32)
            cols.append(jnp.dot(smat, rowsum,
                                precision=lax.Precision.HIGHEST,
                                preferred_element_type=jnp.float32))
        mean = jnp.concatenate(cols, axis=1) * inv_na_ref[...]
        eye = (lax.broadcasted_iota(jnp.int32, (16, 16), 0)
               == lax.broadcasted_iota(jnp.int32, (16, 16), 1)).astype(jnp.float32)
        asig = lax.dot_general(eye, scal_ref[...], (((1,), (1,)), ((), ())),
                               precision=lax.Precision.HIGHEST,
                               preferred_element_type=jnp.float32)
        alpha = asig[:, 0:1]
        table_ref[...] = jnp.concatenate(
            [jnp.sqrt(alpha), jnp.sqrt(1.0 - alpha), asig[:, 1:2], mean,
             jnp.zeros((16, 2), jnp.float32)], axis=1)

    rows = lax.broadcasted_iota(jnp.int32, (16, blk), 1) + i * blk
    oh = ((rows >= starts_ref[...]) & (rows < ends_ref[...])).astype(jnp.float32)
    vals = lax.dot_general(table_ref[...], oh, (((0,), (0,)), ((), ())),
                           precision=lax.Precision.HIGHEST,
                           preferred_element_type=jnp.float32)
    epsr = eps_ref[...] - vals[3:6, :]
    rt_ref[...] = vals[0:1, :] * r_ref[...] + vals[1:2, :] * epsr
    epsr_ref[...] = epsr
    cio = lax.broadcasted_iota(jnp.int32, (_MAXA, blk), 0)
    as_ref[...] = ((cio == (z_ref[...] - 1)).astype(jnp.float32)
                   + comp_ref[...] * vals[2:3, :])


def kernel(r, Z, composition_probs, num_atoms, alphas, type_sigmas):
    N = r.shape[0]
    B = num_atoms.shape[0]
    t = jnp.asarray(_TVALS)
    s = jnp.asarray(_SVALS)

    eps_flat = pl.pallas_call(
        _gen_eps_kernel,
        out_shape=jax.ShapeDtypeStruct((3 * N // 128, 128), jnp.float32),
    )()
    eps_cn = eps_flat.reshape(3, N)

    ends = jnp.cumsum(num_atoms, dtype=jnp.int32)
    starts = ends - num_atoms
    starts_col = starts.reshape(B, 1)
    ends_col = ends.reshape(B, 1)
    inv_na_col = (1.0 / num_atoms.astype(jnp.float32)).reshape(B, 1)
    al_pad = jnp.pad(alphas, (0, 1024 - _T))
    si_pad = jnp.pad(type_sigmas, (0, 1024 - _T))
    starts_p = jnp.pad(starts, (0, 16))
    ends_p = jnp.pad(ends, (0, 16))

    mesh = plsc.VectorSubcoreMesh(core_axis_name="c", subcore_axis_name="s")
    sc_stats = pl.kernel(
        _sc_stats_body,
        out_type=(jax.ShapeDtypeStruct((256, 16), jnp.float32),
                  jax.ShapeDtypeStruct((2, 16), jnp.float32)),
        mesh=mesh,
        compiler_params=pltpu.CompilerParams(needs_layout_passes=False),
        scratch_types=[
            pltpu.VMEM((32,), jnp.int32),
            pltpu.VMEM((32,), jnp.int32),
            pltpu.VMEM((16,), jnp.int32),
            pltpu.VMEM((16,), jnp.int32),
            pltpu.VMEM((1024,), jnp.float32),
            pltpu.VMEM((1024,), jnp.float32),
            pltpu.VMEM((_WIN,), jnp.float32),
            pltpu.VMEM((8, 16), jnp.float32),
            pltpu.VMEM((2, 16), jnp.float32),
        ],
    )
    sums, scal = sc_stats(starts_p, ends_p, t, s, al_pad, si_pad,
                          eps_flat.reshape(-1))

    BLK = 2048
    nb = N // BLK
    rt_t, epsr_t, as_t = pl.pallas_call(
        _main_kernel,
        grid=(nb,),
        in_specs=[
            pl.BlockSpec((B, 1), lambda i: (0, 0)),
            pl.BlockSpec((B, 1), lambda i: (0, 0)),
            pl.BlockSpec((B, 1), lambda i: (0, 0)),
            pl.BlockSpec((256, 16), lambda i: (0, 0)),
            pl.BlockSpec((2, 16), lambda i: (0, 0)),
            pl.BlockSpec((3, BLK), lambda i: (0, i)),
            pl.BlockSpec((3, BLK), lambda i: (0, i)),
            pl.BlockSpec((1, BLK), lambda i: (0, i)),
            pl.BlockSpec((_MAXA, BLK), lambda i: (0, i)),
        ],
        out_specs=[
            pl.BlockSpec((3, BLK), lambda i: (0, i)),
            pl.BlockSpec((3, BLK), lambda i: (0, i)),
            pl.BlockSpec((_MAXA, BLK), lambda i: (0, i)),
        ],
        out_shape=[
            jax.ShapeDtypeStruct((3, N), jnp.float32),
            jax.ShapeDtypeStruct((3, N), jnp.float32),
            jax.ShapeDtypeStruct((_MAXA, N), jnp.float32),
        ],
        scratch_shapes=[pltpu.VMEM((B, 8), jnp.float32)],
    )(starts_col, ends_col, inv_na_col, sums, scal,
      r.T, eps_cn, Z.reshape(1, N), composition_probs.T)

    return (rt_t.T, as_t.T, epsr_t.T, t[:, None], s[:, None])
